# baseline (speedup 1.0000x reference)
import functools

import jax
import jax.numpy as jnp
from jax import lax
from jax.experimental import pallas as pl
from jax.experimental.pallas import tpu as pltpu
from jax.experimental.pallas import tpu_sc as plsc

_F = 768
_H = 128
_ZD = 64
_QD = 32
_NZ = 1024
_NQ = 512
_TOK_BLK = 2048
_N_TOK = 8192
_DP = 128


def _gelu(v):
    return jax.nn.gelu(v)


def _enc_body(x_ref, we1_ref, be1_ref, we2_ref, be2_ref, wz_ref, bz_ref,
              cbzt_ref, wo_ref, borow_ref, ze_ref, zidx_ref, u_ref, acc_ref):
    i = pl.program_id(0)
    x = x_ref[...]
    h = _gelu(jnp.dot(x, we1_ref[...], preferred_element_type=jnp.float32)
              + be1_ref[...])
    h = _gelu(jnp.dot(h, we2_ref[...], preferred_element_type=jnp.float32)
              + be2_ref[...])
    z_e = (jnp.dot(h, wz_ref[...], preferred_element_type=jnp.float32)
           + bz_ref[...])
    cbzt = cbzt_ref[...]
    csq = jnp.sum(cbzt * cbzt, axis=0, keepdims=True)
    zsq = jnp.sum(z_e * z_e, axis=1, keepdims=True)
    d2 = (zsq + csq) - 2.0 * jnp.dot(
        z_e, cbzt, preferred_element_type=jnp.float32)
    minv = jnp.min(d2, axis=1, keepdims=True)
    iota_z = lax.broadcasted_iota(jnp.int32, d2.shape, 1)
    idx_z = jnp.min(jnp.where(d2 == minv, iota_z, _NZ), axis=1,
                    keepdims=True)
    ze_ref[...] = z_e
    zidx_ref[...] = idx_z

    u_ref[...] = jnp.dot(x, wo_ref[...], preferred_element_type=jnp.float32)
    bx = jnp.sum(x * borow_ref[...])
    xx = jnp.sum(x * x)
    lane = lax.broadcasted_iota(jnp.int32, (1, 128), 1)
    vec = jnp.where(lane == 0, bx, 0.0) + jnp.where(lane == 1, xx, 0.0)

    @pl.when(i == 0)
    def _init():
        acc_ref[...] = vec

    @pl.when(i > 0)
    def _accum():
        acc_ref[...] = acc_ref[...] + vec


def _mid_dec_body(u_ref, zqp_ref, ze_ref, wq_ref, bq_ref, cbq_ref, cbqt_ref,
                  wd1_ref, bd1_ref, wd2_ref, bd2_ref, wo_ref, bo_ref,
                  zq_ref, qq_ref, qidx_ref, acc_ref):
    i = pl.program_id(0)
    z_q = zqp_ref[...][:, :_ZD]
    q_e = (jnp.dot(z_q, wq_ref[...], preferred_element_type=jnp.float32)
           + bq_ref[...])
    cbqt = cbqt_ref[...]
    csq_q = jnp.sum(cbqt * cbqt, axis=0, keepdims=True)
    qsq = jnp.sum(q_e * q_e, axis=1, keepdims=True)
    d2q = (qsq + csq_q) - 2.0 * jnp.dot(
        q_e, cbqt, preferred_element_type=jnp.float32)
    minv_q = jnp.min(d2q, axis=1, keepdims=True)
    iota_q = lax.broadcasted_iota(jnp.int32, d2q.shape, 1)
    idx_q = jnp.min(jnp.where(d2q == minv_q, iota_q, _NQ), axis=1,
                    keepdims=True)
    oh_q = (iota_q == idx_q).astype(jnp.float32)
    q_q = jnp.dot(oh_q, cbq_ref[...], preferred_element_type=jnp.float32)

    bf = jnp.bfloat16
    r = _gelu(jnp.dot(q_q.astype(bf), wd1_ref[...].astype(bf),
                      preferred_element_type=jnp.float32) + bd1_ref[...])
    r = _gelu(jnp.dot(r.astype(bf), wd2_ref[...].astype(bf),
                      preferred_element_type=jnp.float32) + bd2_ref[...])
    x_rec = (jnp.dot(r.astype(bf), wo_ref[...].astype(bf),
                     preferred_element_type=jnp.float32) + bo_ref[...])

    dz = z_q - ze_ref[...]
    dq = q_q - q_e
    sq2 = jnp.sum(x_rec * x_rec)
    cross = jnp.sum(r * u_ref[...])
    zs = jnp.sum(dz * dz)
    qs = jnp.sum(dq * dq)

    zq_ref[...] = z_q
    qq_ref[...] = q_q
    qidx_ref[...] = idx_q
    lane = lax.broadcasted_iota(jnp.int32, (1, 128), 1)
    vec = (jnp.where(lane == 0, sq2, 0.0)
           + jnp.where(lane == 1, cross, 0.0)
           + jnp.where(lane == 2, zs, 0.0)
           + jnp.where(lane == 3, qs, 0.0))

    @pl.when(i == 0)
    def _init():
        acc_ref[...] = vec

    @pl.when(i > 0)
    def _accum():
        acc_ref[...] = acc_ref[...] + vec


def _make_sc_gather(n_rows):
    info = plsc.get_sparse_core_info()
    nw = info.num_cores * info.num_subcores
    b_per_w = _N_TOK // nw
    n_chunk = max(1, b_per_w // 32)
    chunk = b_per_w // n_chunk
    mesh = plsc.VectorSubcoreMesh(core_axis_name="c", subcore_axis_name="s")

    @functools.partial(
        pl.kernel, mesh=mesh,
        out_type=jax.ShapeDtypeStruct((_N_TOK, _DP), jnp.float32),
        scratch_types=[
            pltpu.VMEM((b_per_w,), jnp.int32),
            pltpu.VMEM((b_per_w, _DP), jnp.float32),
            pltpu.VMEM_SHARED((n_rows, _DP), jnp.float32),
            pltpu.SemaphoreType.DMA,
        ],
    )
    def g(table_hbm, idx_hbm, out_hbm, idx_v, rows_v, tbl_sh, sem):
        sid = lax.axis_index("s")
        wid = sid * info.num_cores + lax.axis_index("c")
        base = wid * b_per_w

        @pl.when(sid == 0)
        def _stage():
            pltpu.sync_copy(table_hbm, tbl_sh)

        pltpu.sync_copy(idx_hbm.at[pl.ds(base, b_per_w)], idx_v)
        plsc.subcore_barrier()
        cps = [pltpu.async_copy(
                   tbl_sh.at[idx_v.at[pl.ds(c * chunk, chunk)]],
                   rows_v.at[pl.ds(c * chunk, chunk)], sem)
               for c in range(n_chunk)]
        for cp in cps:
            cp.wait()
        pltpu.sync_copy(rows_v, out_hbm.at[pl.ds(base, b_per_w)])

    return g


def kernel(x, W_e1, b_e1, W_e2, b_e2, W_z, b_z, cb_z, W_q, b_q, cb_q,
           W_d1, b_d1, W_d2, b_d2, W_o, b_o):
    B, S, F = x.shape
    N = B * S
    xf = x.reshape(N, F)
    T = _TOK_BLK
    grid = (N // T,)
    full = lambda shape: pl.BlockSpec(shape, lambda i: (0, 0))
    params = pltpu.CompilerParams(dimension_semantics=("arbitrary",))

    cbz_pad = jnp.concatenate(
        [cb_z, jnp.zeros((_NZ, _DP - _ZD), jnp.float32)], axis=1)

    z_e, z_idx, u, parts_a = pl.pallas_call(
        _enc_body,
        grid=grid,
        in_specs=[
            pl.BlockSpec((T, F), lambda i: (i, 0)),
            full((F, 64)), full((1, 64)),
            full((64, _H)), full((1, _H)),
            full((_H, _ZD)), full((1, _ZD)),
            full((_ZD, _NZ)),
            full((F, _H)), full((1, F)),
        ],
        out_specs=(
            pl.BlockSpec((T, _ZD), lambda i: (i, 0)),
            pl.BlockSpec((T, 1), lambda i: (i, 0)),
            pl.BlockSpec((T, _H), lambda i: (i, 0)),
            pl.BlockSpec((1, 128), lambda i: (0, 0)),
        ),
        out_shape=(
            jax.ShapeDtypeStruct((N, _ZD), jnp.float32),
            jax.ShapeDtypeStruct((N, 1), jnp.int32),
            jax.ShapeDtypeStruct((N, _H), jnp.float32),
            jax.ShapeDtypeStruct((1, 128), jnp.float32),
        ),
        compiler_params=params,
    )(xf, W_e1.T, b_e1[None, :], W_e2.T, b_e2[None, :], W_z.T, b_z[None, :],
      cb_z.T, W_o, b_o[None, :])

    z_q_pad = _make_sc_gather(_NZ)(cbz_pad, z_idx.reshape(N))

    z_q, q_q, q_idx, parts = pl.pallas_call(
        _mid_dec_body,
        grid=grid,
        in_specs=[
            pl.BlockSpec((T, _H), lambda i: (i, 0)),
            pl.BlockSpec((T, _DP), lambda i: (i, 0)),
            pl.BlockSpec((T, _ZD), lambda i: (i, 0)),
            full((_ZD, _QD)), full((1, _QD)),
            full((_NQ, _QD)), full((_QD, _NQ)),
            full((_QD, 64)), full((1, 64)),
            full((64, _H)), full((1, _H)),
            full((_H, _F)), full((1, _F)),
        ],
        out_specs=(
            pl.BlockSpec((T, _ZD), lambda i: (i, 0)),
            pl.BlockSpec((T, _QD), lambda i: (i, 0)),
            pl.BlockSpec((T, 1), lambda i: (i, 0)),
            pl.BlockSpec((1, 128), lambda i: (0, 0)),
        ),
        out_shape=(
            jax.ShapeDtypeStruct((N, _ZD), jnp.float32),
            jax.ShapeDtypeStruct((N, _QD), jnp.float32),
            jax.ShapeDtypeStruct((N, 1), jnp.int32),
            jax.ShapeDtypeStruct((1, 128), jnp.float32),
        ),
        compiler_params=params,
    )(u, z_q_pad, z_e, W_q.T, b_q[None, :], cb_q, cb_q.T,
      W_d1.T, b_d1[None, :], W_d2.T, b_d2[None, :], W_o.T, b_o[None, :])

    recon_sum = (parts[0, 0]
                 - 2.0 * (parts[0, 1] + parts_a[0, 0]) + parts_a[0, 1])
    loss = (recon_sum / (N * _F)
            + 0.5 * (parts[0, 2] / (N * _ZD) + parts[0, 3] / (N * _QD)))
    return (z_q.reshape(B, S, _ZD), q_q.reshape(B, S, _QD),
            z_idx.reshape(B, S), q_idx.reshape(B, S), loss)


# speedup vs baseline: 1.0762x; 1.0762x over previous
import functools

import jax
import jax.numpy as jnp
from jax import lax
from jax.experimental import pallas as pl
from jax.experimental.pallas import tpu as pltpu
from jax.experimental.pallas import tpu_sc as plsc

_F = 768
_H = 128
_ZD = 64
_QD = 32
_NZ = 1024
_NQ = 512
_TOK_BLK = 2048
_N_TOK = 8192
_DP = 128


def _gelu(v):
    return jax.nn.gelu(v)


def _enc_body(x_ref, we1_ref, be1_ref, we2_ref, be2_ref, wz_ref, bz_ref,
              cbzt_ref, ze_ref, zidx_ref):
    x = x_ref[...]
    h = _gelu(jnp.dot(x, we1_ref[...], preferred_element_type=jnp.float32)
              + be1_ref[...])
    h = _gelu(jnp.dot(h, we2_ref[...], preferred_element_type=jnp.float32)
              + be2_ref[...])
    z_e = (jnp.dot(h, wz_ref[...], preferred_element_type=jnp.float32)
           + bz_ref[...])
    cbzt = cbzt_ref[...]
    csq = jnp.sum(cbzt * cbzt, axis=0, keepdims=True)
    zsq = jnp.sum(z_e * z_e, axis=1, keepdims=True)
    d2 = (zsq + csq) - 2.0 * jnp.dot(
        z_e, cbzt, preferred_element_type=jnp.float32)
    minv = jnp.min(d2, axis=1, keepdims=True)
    iota_z = lax.broadcasted_iota(jnp.int32, d2.shape, 1)
    idx_z = jnp.min(jnp.where(d2 == minv, iota_z, _NZ), axis=1,
                    keepdims=True)
    ze_ref[...] = z_e
    zidx_ref[...] = idx_z


def _mid_dec_body(x_ref, zqp_ref, ze_ref, wq_ref, bq_ref, cbq_ref, cbqt_ref,
                  wd1_ref, bd1_ref, wd2_ref, bd2_ref, wo_ref, bo_ref,
                  zq_ref, qq_ref, qidx_ref, acc_ref):
    i = pl.program_id(0)
    z_q = zqp_ref[...]
    q_e = (jnp.dot(z_q, wq_ref[...], preferred_element_type=jnp.float32)
           + bq_ref[...])
    cbqt = cbqt_ref[...]
    csq_q = jnp.sum(cbqt * cbqt, axis=0, keepdims=True)
    qsq = jnp.sum(q_e * q_e, axis=1, keepdims=True)
    d2q = (qsq + csq_q) - 2.0 * jnp.dot(
        q_e, cbqt, preferred_element_type=jnp.float32)
    minv_q = jnp.min(d2q, axis=1, keepdims=True)
    iota_q = lax.broadcasted_iota(jnp.int32, d2q.shape, 1)
    idx_q = jnp.min(jnp.where(d2q == minv_q, iota_q, _NQ), axis=1,
                    keepdims=True)
    oh_q = (iota_q == idx_q).astype(jnp.float32)
    q_q = jnp.dot(oh_q, cbq_ref[...], preferred_element_type=jnp.float32)

    bf = jnp.bfloat16
    r = _gelu(jnp.dot(q_q.astype(bf), wd1_ref[...].astype(bf),
                      preferred_element_type=jnp.float32) + bd1_ref[...])
    r = _gelu(jnp.dot(r.astype(bf), wd2_ref[...].astype(bf),
                      preferred_element_type=jnp.float32) + bd2_ref[...])
    x_rec = (jnp.dot(r.astype(bf), wo_ref[...].astype(bf),
                     preferred_element_type=jnp.float32) + bo_ref[...])

    dr = x_rec - x_ref[...]
    dz = z_q - ze_ref[...]
    dq = q_q - q_e
    rs = jnp.sum(dr * dr)
    zs = jnp.sum(dz * dz)
    qs = jnp.sum(dq * dq)

    zq_ref[...] = z_q
    qq_ref[...] = q_q
    qidx_ref[...] = idx_q
    lane = lax.broadcasted_iota(jnp.int32, (1, 128), 1)
    vec = (jnp.where(lane == 0, rs, 0.0)
           + jnp.where(lane == 1, zs, 0.0)
           + jnp.where(lane == 2, qs, 0.0))

    @pl.when(i == 0)
    def _init():
        acc_ref[...] = vec

    @pl.when(i > 0)
    def _accum():
        acc_ref[...] = acc_ref[...] + vec


def _make_sc_gather(n_rows):
    info = plsc.get_sparse_core_info()
    nw = info.num_cores * info.num_subcores
    b_per_w = _N_TOK // nw
    n_chunk = max(1, b_per_w // 32)
    chunk = b_per_w // n_chunk
    mesh = plsc.VectorSubcoreMesh(core_axis_name="c", subcore_axis_name="s")

    @functools.partial(
        pl.kernel, mesh=mesh,
        out_type=jax.ShapeDtypeStruct((_N_TOK, _ZD), jnp.float32),
        scratch_types=[
            pltpu.VMEM((b_per_w,), jnp.int32),
            pltpu.VMEM((b_per_w, _ZD), jnp.float32),
            pltpu.VMEM_SHARED((n_rows, _ZD), jnp.float32),
            pltpu.SemaphoreType.DMA,
        ],
    )
    def g(table_hbm, idx_hbm, out_hbm, idx_v, rows_v, tbl_sh, sem):
        sid = lax.axis_index("s")
        wid = sid * info.num_cores + lax.axis_index("c")
        base = wid * b_per_w

        @pl.when(sid == 0)
        def _stage():
            pltpu.sync_copy(table_hbm, tbl_sh)

        pltpu.sync_copy(idx_hbm.at[pl.ds(base, b_per_w)], idx_v)
        plsc.subcore_barrier()
        cps = [pltpu.async_copy(
                   tbl_sh.at[idx_v.at[pl.ds(c * chunk, chunk)]],
                   rows_v.at[pl.ds(c * chunk, chunk)], sem)
               for c in range(n_chunk)]
        for cp in cps:
            cp.wait()
        pltpu.sync_copy(rows_v, out_hbm.at[pl.ds(base, b_per_w)])

    return g


def kernel(x, W_e1, b_e1, W_e2, b_e2, W_z, b_z, cb_z, W_q, b_q, cb_q,
           W_d1, b_d1, W_d2, b_d2, W_o, b_o):
    B, S, F = x.shape
    N = B * S
    xf = x.reshape(N, F)
    T = _TOK_BLK
    grid = (N // T,)
    full = lambda shape: pl.BlockSpec(shape, lambda i: (0, 0))
    params = pltpu.CompilerParams(dimension_semantics=("arbitrary",))

    z_e, z_idx = pl.pallas_call(
        _enc_body,
        grid=grid,
        in_specs=[
            pl.BlockSpec((T, F), lambda i: (i, 0)),
            full((F, 64)), full((1, 64)),
            full((64, _H)), full((1, _H)),
            full((_H, _ZD)), full((1, _ZD)),
            full((_ZD, _NZ)),
        ],
        out_specs=(
            pl.BlockSpec((T, _ZD), lambda i: (i, 0)),
            pl.BlockSpec((T, 1), lambda i: (i, 0)),
        ),
        out_shape=(
            jax.ShapeDtypeStruct((N, _ZD), jnp.float32),
            jax.ShapeDtypeStruct((N, 1), jnp.int32),
        ),
        compiler_params=params,
    )(xf, W_e1.T, b_e1[None, :], W_e2.T, b_e2[None, :], W_z.T, b_z[None, :],
      cb_z.T)

    z_q_sc = _make_sc_gather(_NZ)(cb_z, z_idx.reshape(N))

    z_q, q_q, q_idx, parts = pl.pallas_call(
        _mid_dec_body,
        grid=grid,
        in_specs=[
            pl.BlockSpec((T, F), lambda i: (i, 0)),
            pl.BlockSpec((T, _ZD), lambda i: (i, 0)),
            pl.BlockSpec((T, _ZD), lambda i: (i, 0)),
            full((_ZD, _QD)), full((1, _QD)),
            full((_NQ, _QD)), full((_QD, _NQ)),
            full((_QD, 64)), full((1, 64)),
            full((64, _H)), full((1, _H)),
            full((_H, _F)), full((1, _F)),
        ],
        out_specs=(
            pl.BlockSpec((T, _ZD), lambda i: (i, 0)),
            pl.BlockSpec((T, _QD), lambda i: (i, 0)),
            pl.BlockSpec((T, 1), lambda i: (i, 0)),
            pl.BlockSpec((1, 128), lambda i: (0, 0)),
        ),
        out_shape=(
            jax.ShapeDtypeStruct((N, _ZD), jnp.float32),
            jax.ShapeDtypeStruct((N, _QD), jnp.float32),
            jax.ShapeDtypeStruct((N, 1), jnp.int32),
            jax.ShapeDtypeStruct((1, 128), jnp.float32),
        ),
        compiler_params=params,
    )(xf, z_q_sc, z_e, W_q.T, b_q[None, :], cb_q, cb_q.T,
      W_d1.T, b_d1[None, :], W_d2.T, b_d2[None, :], W_o.T, b_o[None, :])

    loss = (parts[0, 0] / (N * _F)
            + 0.5 * (parts[0, 1] / (N * _ZD) + parts[0, 2] / (N * _QD)))
    return (z_q.reshape(B, S, _ZD), q_q.reshape(B, S, _QD),
            z_idx.reshape(B, S), q_idx.reshape(B, S), loss)
